# baseline (speedup 1.0000x reference)
import jax
import jax.numpy as jnp
from jax.experimental import pallas as pl
from jax.experimental.pallas import tpu as pltpu

_N = 4096
_NFEAT = 128
_D1 = 32
_D2 = 32
_NMETA = 3
_HOUSE = 512
_SEQ = _N // _HOUSE
_RS = 512
_NSTRIP = _N // _RS


def _dot(a, b):
    return jnp.dot(a, b, preferred_element_type=jnp.float32)


def _b16(t):
    return t.astype(jnp.bfloat16)


def _fused_kernel(w_ref, adj_ref, x_ref, gc1_w_ref, gc1_b_ref, gc2_w_ref,
                  gc2_b_ref, w_ih_t_ref, w_hh_t_ref, b_ih_ref, b_hh_ref,
                  lin_w_t_ref, lin_b_ref, out_ref,
                  ub_sc, fz_sc, stash_sc):
    i = pl.program_id(0)
    r = pl.program_id(1)
    rows = pl.ds(r * _RS, _RS)

    @pl.when((i == 0) & (r == 0))
    def _init():
        ub_sc[:, 0:_D1] = _b16(_dot(x_ref[...], gc1_w_ref[...]))

    @pl.when((i > 0) & (r == 0))
    def _promote_v():
        ub_sc[:, _D1:_D1 + _D2] = _b16(fz_sc[:, 0:_D2])

    @pl.when(i == 0)
    def _fill_meta0():
        abf = _b16(adj_ref[0])
        y1 = _dot(abf, ub_sc[:, 0:_D1])
        v = _dot(jnp.maximum(y1 + gc1_b_ref[...], 0.0), gc2_w_ref[...])
        fz_sc[rows, 0:_D2] = v
        stash_sc[rows, :] = abf

    @pl.when((i > 0) & (i < _NMETA))
    def _mid():
        abf = _b16(adj_ref[0])
        y1 = _dot(abf, ub_sc[:, 0:_D1])
        y2 = _dot(stash_sc[rows, :], ub_sc[:, _D1:_D1 + _D2])
        v = _dot(jnp.maximum(y1 + gc1_b_ref[...], 0.0), gc2_w_ref[...])
        h2 = jnp.maximum(y2 + gc2_b_ref[...], 0.0)
        contrib = w_ref[i - 1, 0] * h2
        fz_sc[rows, 0:_D2] = v
        fz_sc[rows, _D2:2 * _D2] = jnp.where(
            i == 1, contrib, fz_sc[rows, _D2:2 * _D2] + contrib)
        stash_sc[rows, :] = abf

    @pl.when((i == _NMETA) & (r < 2))
    def _layer2_drain():
        half = pl.ds(r * (_N // 2), _N // 2)
        y2 = _dot(stash_sc[half, :], ub_sc[:, _D1:_D1 + _D2])
        h2 = jnp.maximum(y2 + gc2_b_ref[...], 0.0)
        fz_sc[half, _D2:2 * _D2] = (fz_sc[half, _D2:2 * _D2]
                                    + w_ref[_NMETA - 1, 0] * h2)

    @pl.when((i == _NMETA) & (r == 2))
    def _lstm_and_linear():
        fz_sc[:, _D2:2 * _D2] = jnp.maximum(fz_sc[:, _D2:2 * _D2], 0.0)
        w_ih_t = w_ih_t_ref[...]
        w_hh_t = w_hh_t_ref[...]
        b = b_ih_ref[...] + b_hh_ref[...]
        lin_w_t = lin_w_t_ref[...]
        lin_b = lin_b_ref[...]

        def step(t, carry):
            h, cc = carry
            seq_rows = pl.ds(t * _HOUSE, _HOUSE)
            gates = (_dot(fz_sc[seq_rows, _D2:2 * _D2], w_ih_t)
                     + _dot(h, w_hh_t) + b)
            ig = jax.nn.sigmoid(gates[:, 0 * _D2:1 * _D2])
            fg = jax.nn.sigmoid(gates[:, 1 * _D2:2 * _D2])
            gg = jnp.tanh(gates[:, 2 * _D2:3 * _D2])
            og = jax.nn.sigmoid(gates[:, 3 * _D2:4 * _D2])
            c_new = fg * cc + ig * gg
            h_new = og * jnp.tanh(c_new)
            out_ref[seq_rows, :] = _dot(h_new, lin_w_t) + lin_b
            return h_new, c_new

        h0 = jnp.zeros((_HOUSE, _D2), dtype=jnp.float32)
        c0 = jnp.zeros((_HOUSE, _D2), dtype=jnp.float32)
        jax.lax.fori_loop(0, _SEQ, step, (h0, c0))


def _adj_index(i, r):
    return (jnp.minimum(i, _NMETA - 1),
            jnp.where(i < _NMETA, r, _NSTRIP - 1), 0)


def kernel(adj, x, W, gc1_w, gc1_b, gc2_w, gc2_b, w_ih, w_hh, b_ih, b_hh,
           lin_w, lin_b):
    grid = (_NMETA + 1, _NSTRIP)
    out = pl.pallas_call(
        _fused_kernel,
        grid=grid,
        in_specs=[
            pl.BlockSpec(memory_space=pltpu.SMEM),
            pl.BlockSpec((1, _RS, _N), _adj_index),
            pl.BlockSpec((_N, _NFEAT), lambda i, r: (0, 0)),
            pl.BlockSpec((_NFEAT, _D1), lambda i, r: (0, 0)),
            pl.BlockSpec((1, _D1), lambda i, r: (0, 0)),
            pl.BlockSpec((_D1, _D2), lambda i, r: (0, 0)),
            pl.BlockSpec((1, _D2), lambda i, r: (0, 0)),
            pl.BlockSpec((_D2, 4 * _D2), lambda i, r: (0, 0)),
            pl.BlockSpec((_D2, 4 * _D2), lambda i, r: (0, 0)),
            pl.BlockSpec((1, 4 * _D2), lambda i, r: (0, 0)),
            pl.BlockSpec((1, 4 * _D2), lambda i, r: (0, 0)),
            pl.BlockSpec((_D2, 1), lambda i, r: (0, 0)),
            pl.BlockSpec((1, 1), lambda i, r: (0, 0)),
        ],
        out_specs=pl.BlockSpec((_N, 1), lambda i, r: (0, 0)),
        out_shape=jax.ShapeDtypeStruct((_N, 1), jnp.float32),
        scratch_shapes=[
            pltpu.VMEM((_N, _D1 + _D2), jnp.bfloat16),
            pltpu.VMEM((_N, 2 * _D2), jnp.float32),
            pltpu.VMEM((_N, _N), jnp.bfloat16),
        ],
        compiler_params=pltpu.CompilerParams(
            dimension_semantics=("arbitrary", "arbitrary"),
            vmem_limit_bytes=100 * 1024 * 1024,
        ),
    )(W, adj, x, gc1_w, gc1_b.reshape(1, _D1), gc2_w,
      gc2_b.reshape(1, _D2), w_ih.T, w_hh.T, b_ih.reshape(1, 4 * _D2),
      b_hh.reshape(1, 4 * _D2), lin_w.T, lin_b.reshape(1, 1))
    return out


# speedup vs baseline: 1.2865x; 1.2865x over previous
import jax
import jax.numpy as jnp
from jax.experimental import pallas as pl
from jax.experimental.pallas import tpu as pltpu

_N = 4096
_NFEAT = 128
_D1 = 32
_D2 = 32
_NMETA = 3
_HOUSE = 512
_SEQ = _N // _HOUSE
_RS = 512
_NSTRIP = _N // _RS


def _dot(a, b):
    return jnp.dot(a, b, preferred_element_type=jnp.float32)


def _b16(t):
    return t.astype(jnp.bfloat16)


def _fused_kernel(w_ref, adj_ref, x_ref, gc1_w_ref, gc1_b_ref, gc2_w_ref,
                  gc2_b_ref, w_ih_t_ref, w_hh_t_ref, b_ih_ref, b_hh_ref,
                  lin_w_t_ref, lin_b_ref, out_ref,
                  u_sc, vp_sc, vc_sc, z_sc, stash_sc):
    i = pl.program_id(0)
    r = pl.program_id(1)
    rows = pl.ds(r * _RS, _RS)

    @pl.when((i == 0) & (r == 0))
    def _init():
        u_sc[...] = _b16(_dot(x_ref[...], gc1_w_ref[...]))

    @pl.when((i > 0) & (r == 0))
    def _promote_v():
        vp_sc[...] = _b16(vc_sc[...])

    @pl.when((i > 0) & (i < _NMETA))
    def _layer2_prev_meta():
        y2 = _dot(stash_sc[rows, :], vp_sc[...])
        h2 = jnp.maximum(y2 + gc2_b_ref[...], 0.0)
        contrib = w_ref[i - 1, 0] * h2

        @pl.when(i == 1)
        def _():
            z_sc[rows, :] = contrib

        @pl.when(i > 1)
        def _():
            z_sc[rows, :] = z_sc[rows, :] + contrib

    @pl.when((i == _NMETA) & (r < 2))
    def _layer2_drain():
        half = pl.ds(r * (_N // 2), _N // 2)
        y2 = _dot(stash_sc[half, :], vp_sc[...])
        h2 = jnp.maximum(y2 + gc2_b_ref[...], 0.0)
        z_sc[half, :] = z_sc[half, :] + w_ref[_NMETA - 1, 0] * h2

    @pl.when(i < _NMETA)
    def _layer1_cur_meta():
        abf = _b16(adj_ref[0])
        y1 = _dot(abf, u_sc[...])
        v = _dot(jnp.maximum(y1 + gc1_b_ref[...], 0.0), gc2_w_ref[...])
        vc_sc[rows, :] = v
        stash_sc[rows, :] = abf

    @pl.when((i == _NMETA) & (r == 2))
    def _lstm_and_linear():
        z_sc[...] = jnp.maximum(z_sc[...], 0.0)
        w_ih_t = w_ih_t_ref[...]
        w_hh_t = w_hh_t_ref[...]
        b = b_ih_ref[...] + b_hh_ref[...]
        lin_w_t = lin_w_t_ref[...]
        lin_b = lin_b_ref[...]

        def step(t, carry):
            h, cc = carry
            seq_rows = pl.ds(t * _HOUSE, _HOUSE)
            gates = _dot(z_sc[seq_rows, :], w_ih_t) + _dot(h, w_hh_t) + b
            ig = jax.nn.sigmoid(gates[:, 0 * _D2:1 * _D2])
            fg = jax.nn.sigmoid(gates[:, 1 * _D2:2 * _D2])
            gg = jnp.tanh(gates[:, 2 * _D2:3 * _D2])
            og = jax.nn.sigmoid(gates[:, 3 * _D2:4 * _D2])
            c_new = fg * cc + ig * gg
            h_new = og * jnp.tanh(c_new)
            out_ref[seq_rows, :] = _dot(h_new, lin_w_t) + lin_b
            return h_new, c_new

        h0 = jnp.zeros((_HOUSE, _D2), dtype=jnp.float32)
        c0 = jnp.zeros((_HOUSE, _D2), dtype=jnp.float32)
        jax.lax.fori_loop(0, _SEQ, step, (h0, c0))


def _adj_index(i, r):
    return (jnp.minimum(i, _NMETA - 1),
            jnp.where(i < _NMETA, r, _NSTRIP - 1), 0)


def kernel(adj, x, W, gc1_w, gc1_b, gc2_w, gc2_b, w_ih, w_hh, b_ih, b_hh,
           lin_w, lin_b):
    grid = (_NMETA + 1, _NSTRIP)
    out = pl.pallas_call(
        _fused_kernel,
        grid=grid,
        in_specs=[
            pl.BlockSpec(memory_space=pltpu.SMEM),
            pl.BlockSpec((1, _RS, _N), _adj_index),
            pl.BlockSpec((_N, _NFEAT), lambda i, r: (0, 0)),
            pl.BlockSpec((_NFEAT, _D1), lambda i, r: (0, 0)),
            pl.BlockSpec((1, _D1), lambda i, r: (0, 0)),
            pl.BlockSpec((_D1, _D2), lambda i, r: (0, 0)),
            pl.BlockSpec((1, _D2), lambda i, r: (0, 0)),
            pl.BlockSpec((_D2, 4 * _D2), lambda i, r: (0, 0)),
            pl.BlockSpec((_D2, 4 * _D2), lambda i, r: (0, 0)),
            pl.BlockSpec((1, 4 * _D2), lambda i, r: (0, 0)),
            pl.BlockSpec((1, 4 * _D2), lambda i, r: (0, 0)),
            pl.BlockSpec((_D2, 1), lambda i, r: (0, 0)),
            pl.BlockSpec((1, 1), lambda i, r: (0, 0)),
        ],
        out_specs=pl.BlockSpec((_N, 1), lambda i, r: (0, 0)),
        out_shape=jax.ShapeDtypeStruct((_N, 1), jnp.float32),
        scratch_shapes=[
            pltpu.VMEM((_N, _D1), jnp.bfloat16),
            pltpu.VMEM((_N, _D2), jnp.bfloat16),
            pltpu.VMEM((_N, _D2), jnp.float32),
            pltpu.VMEM((_N, _D2), jnp.float32),
            pltpu.VMEM((_N, _N), jnp.bfloat16),
        ],
        compiler_params=pltpu.CompilerParams(
            dimension_semantics=("arbitrary", "arbitrary"),
            vmem_limit_bytes=100 * 1024 * 1024,
        ),
    )(W, adj, x, gc1_w, gc1_b.reshape(1, _D1), gc2_w,
      gc2_b.reshape(1, _D2), w_ih.T, w_hh.T, b_ih.reshape(1, 4 * _D2),
      b_hh.reshape(1, 4 * _D2), lin_w.T, lin_b.reshape(1, 1))
    return out
